# baseline (speedup 1.0000x reference)
import jax
import jax.numpy as jnp
from jax import lax
from jax.experimental import pallas as pl
from jax.experimental.pallas import tpu as pltpu
from jax.experimental.pallas import tpu_sc as plsc

NUM_ROWS = 1_000_000
DIM = 64
BATCH = 425_984

NC = 2
NS = 16
NW = NC * NS
CHUNK = 128
ROWS_PER_W = BATCH // NW
CHUNKS_PER_W = ROWS_PER_W // CHUNK


def _gather_body(table_hbm, idx_hbm, out_hbm, idx_v, rows_v, sem):
    wid = lax.axis_index("s") * NC + lax.axis_index("c")
    pltpu.sync_copy(idx_hbm.at[pl.ds(wid * CHUNKS_PER_W, CHUNKS_PER_W)], idx_v)
    base = wid * ROWS_PER_W

    @pl.loop(0, CHUNKS_PER_W)
    def _chunk(j):
        pltpu.async_copy(table_hbm.at[idx_v.at[j]], rows_v, sem).wait()
        pltpu.sync_copy(rows_v, out_hbm.at[pl.ds(base + j * CHUNK, CHUNK)])


def kernel(tensor, index):
    idx2d = index.astype(jnp.int32).reshape(BATCH // CHUNK, CHUNK)
    mesh = plsc.VectorSubcoreMesh(core_axis_name="c", subcore_axis_name="s")
    k = pl.kernel(
        _gather_body,
        out_type=jax.ShapeDtypeStruct((BATCH, DIM), jnp.float32),
        mesh=mesh,
        scratch_types=[
            pltpu.VMEM((CHUNKS_PER_W, CHUNK), jnp.int32),
            pltpu.VMEM((CHUNK, DIM), jnp.float32),
            pltpu.SemaphoreType.DMA,
        ],
        compiler_params=pltpu.CompilerParams(use_tc_tiling_on_sc=False),
    )
    return k(tensor, idx2d)


# speedup vs baseline: 1.0802x; 1.0802x over previous
import jax
import jax.numpy as jnp
from jax import lax
from jax.experimental import pallas as pl
from jax.experimental.pallas import tpu as pltpu
from jax.experimental.pallas import tpu_sc as plsc

NUM_ROWS = 1_000_000
DIM = 64
BATCH = 425_984

NC = 2
NS = 16
NW = NC * NS
CHUNK = 128
ROWS_PER_W = BATCH // NW
CHUNKS_PER_W = ROWS_PER_W // CHUNK


NBUF = 8
NGROUPS = CHUNKS_PER_W // NBUF


def _gather_body(table_hbm, idx_hbm, out_hbm, idx_v, rows_v, gsem, wsem):
    wid = lax.axis_index("s") * NC + lax.axis_index("c")
    pltpu.sync_copy(idx_hbm.at[pl.ds(wid * CHUNKS_PER_W, CHUNKS_PER_W)], idx_v)
    base = wid * ROWS_PER_W

    for b in range(NBUF):
        pltpu.async_copy(table_hbm.at[idx_v.at[b]], rows_v.at[b], gsem.at[b])

    @pl.loop(0, NGROUPS - 1)
    def _group(g):
        for b in range(NBUF):
            j = g * NBUF + b
            pltpu.make_async_copy(
                table_hbm.at[idx_v.at[j]], rows_v.at[b], gsem.at[b]
            ).wait()
            pltpu.async_copy(
                rows_v.at[b],
                out_hbm.at[pl.ds(base + j * CHUNK, CHUNK)],
                wsem.at[b],
            ).wait()
            pltpu.async_copy(
                table_hbm.at[idx_v.at[j + NBUF]], rows_v.at[b], gsem.at[b]
            )

    for b in range(NBUF):
        j = (NGROUPS - 1) * NBUF + b
        pltpu.make_async_copy(
            table_hbm.at[idx_v.at[j]], rows_v.at[b], gsem.at[b]
        ).wait()
        pltpu.sync_copy(rows_v.at[b], out_hbm.at[pl.ds(base + j * CHUNK, CHUNK)])


def kernel(tensor, index):
    idx2d = index.astype(jnp.int32).reshape(BATCH // CHUNK, CHUNK)
    mesh = plsc.VectorSubcoreMesh(core_axis_name="c", subcore_axis_name="s")
    k = pl.kernel(
        _gather_body,
        out_type=jax.ShapeDtypeStruct((BATCH, DIM), jnp.float32),
        mesh=mesh,
        scratch_types=[
            pltpu.VMEM((CHUNKS_PER_W, CHUNK), jnp.int32),
            pltpu.VMEM((NBUF, CHUNK, DIM), jnp.float32),
            pltpu.SemaphoreType.DMA((NBUF,)),
            pltpu.SemaphoreType.DMA((NBUF,)),
        ],
        compiler_params=pltpu.CompilerParams(use_tc_tiling_on_sc=False),
    )
    return k(tensor, idx2d)


# speedup vs baseline: 1.2826x; 1.1874x over previous
import jax
import jax.numpy as jnp
from jax import lax
from jax.experimental import pallas as pl
from jax.experimental.pallas import tpu as pltpu
from jax.experimental.pallas import tpu_sc as plsc

NUM_ROWS = 1_000_000
DIM = 64
BATCH = 425_984

NC = 2
NS = 16
NW = NC * NS
CHUNK = 128
DIMP = 128
ROWS_PER_W = BATCH // NW
CHUNKS_PER_W = ROWS_PER_W // CHUNK


NBUF = 4
NGROUPS = CHUNKS_PER_W // NBUF


def _gather_body(table_hbm, idx_hbm, out_hbm, idx_v, rows_v, gsem, wsem):
    wid = lax.axis_index("s") * NC + lax.axis_index("c")
    pltpu.sync_copy(idx_hbm.at[pl.ds(wid * CHUNKS_PER_W, CHUNKS_PER_W)], idx_v)
    base = wid * ROWS_PER_W

    for b in range(NBUF):
        pltpu.async_copy(table_hbm.at[idx_v.at[b]], rows_v.at[b], gsem.at[b])

    @pl.loop(0, NGROUPS - 1)
    def _group(g):
        for b in range(NBUF):
            j = g * NBUF + b
            pltpu.make_async_copy(
                table_hbm.at[idx_v.at[j]], rows_v.at[b], gsem.at[b]
            ).wait()
            pltpu.async_copy(
                rows_v.at[b],
                out_hbm.at[pl.ds(base + j * CHUNK, CHUNK)],
                wsem.at[b],
            ).wait()
            pltpu.async_copy(
                table_hbm.at[idx_v.at[j + NBUF]], rows_v.at[b], gsem.at[b]
            )

    for b in range(NBUF):
        j = (NGROUPS - 1) * NBUF + b
        pltpu.make_async_copy(
            table_hbm.at[idx_v.at[j]], rows_v.at[b], gsem.at[b]
        ).wait()
        pltpu.sync_copy(rows_v.at[b], out_hbm.at[pl.ds(base + j * CHUNK, CHUNK)])


def kernel(tensor, index):
    tpad = jnp.pad(tensor, ((0, 0), (0, DIMP - DIM)))
    idx2d = index.astype(jnp.int32).reshape(BATCH // CHUNK, CHUNK)
    mesh = plsc.VectorSubcoreMesh(core_axis_name="c", subcore_axis_name="s")
    k = pl.kernel(
        _gather_body,
        out_type=jax.ShapeDtypeStruct((BATCH, DIMP), jnp.float32),
        mesh=mesh,
        scratch_types=[
            pltpu.VMEM((CHUNKS_PER_W, CHUNK), jnp.int32),
            pltpu.VMEM((NBUF, CHUNK, DIMP), jnp.float32),
            pltpu.SemaphoreType.DMA((NBUF,)),
            pltpu.SemaphoreType.DMA((NBUF,)),
        ],
        compiler_params=pltpu.CompilerParams(use_tc_tiling_on_sc=False),
    )
    out128 = k(tpad, idx2d)
    return out128[:, :DIM]


# speedup vs baseline: 1.4012x; 1.0924x over previous
import jax
import jax.numpy as jnp
from jax import lax
from jax.experimental import pallas as pl
from jax.experimental.pallas import tpu as pltpu
from jax.experimental.pallas import tpu_sc as plsc

NUM_ROWS = 1_000_000
DIM = 64
BATCH = 425_984

NC = 2
NS = 16
NW = NC * NS
CHUNK = 128
DIMP = 128
ROWS_PER_W = BATCH // NW
CHUNKS_PER_W = ROWS_PER_W // CHUNK


NBUF = 4
NGROUPS = CHUNKS_PER_W // NBUF


def _gather_body(table_hbm, idx_hbm, out_hbm, idx_v, rows_v, gsem, wsem):
    wid = lax.axis_index("s") * NC + lax.axis_index("c")
    pltpu.sync_copy(idx_hbm.at[pl.ds(wid * CHUNKS_PER_W, CHUNKS_PER_W)], idx_v)
    base = wid * ROWS_PER_W

    for b in range(NBUF):
        pltpu.async_copy(table_hbm.at[idx_v.at[b]], rows_v.at[b], gsem.at[b])

    @pl.loop(0, NGROUPS - 1)
    def _group(g):
        for b in range(NBUF):
            j = g * NBUF + b
            pltpu.make_async_copy(
                table_hbm.at[idx_v.at[j]], rows_v.at[b], gsem.at[b]
            ).wait()
            pltpu.async_copy(
                rows_v.at[b],
                out_hbm.at[pl.ds(base + j * CHUNK, CHUNK)],
                wsem.at[b],
            ).wait()
            pltpu.async_copy(
                table_hbm.at[idx_v.at[j + NBUF]], rows_v.at[b], gsem.at[b]
            )

    for b in range(NBUF):
        j = (NGROUPS - 1) * NBUF + b
        pltpu.make_async_copy(
            table_hbm.at[idx_v.at[j]], rows_v.at[b], gsem.at[b]
        ).wait()
        pltpu.sync_copy(rows_v.at[b], out_hbm.at[pl.ds(base + j * CHUNK, CHUNK)])


TR_BLOCK = 2048
TR_GRID = (NUM_ROWS + TR_BLOCK - 1) // TR_BLOCK


def _transpose_body(in_ref, out_ref):
    out_ref[:, :DIM] = in_ref[...].T


def _transpose_pad(tT):
    return pl.pallas_call(
        _transpose_body,
        grid=(TR_GRID,),
        in_specs=[pl.BlockSpec((DIM, TR_BLOCK), lambda i: (0, i))],
        out_specs=pl.BlockSpec((TR_BLOCK, DIMP), lambda i: (i, 0)),
        out_shape=jax.ShapeDtypeStruct((NUM_ROWS, DIMP), jnp.float32),
    )(tT)


def kernel(tensor, index):
    tpad = _transpose_pad(jnp.swapaxes(tensor, 0, 1))
    idx2d = index.astype(jnp.int32).reshape(BATCH // CHUNK, CHUNK)
    mesh = plsc.VectorSubcoreMesh(core_axis_name="c", subcore_axis_name="s")
    k = pl.kernel(
        _gather_body,
        out_type=jax.ShapeDtypeStruct((BATCH, DIMP), jnp.float32),
        mesh=mesh,
        scratch_types=[
            pltpu.VMEM((CHUNKS_PER_W, CHUNK), jnp.int32),
            pltpu.VMEM((NBUF, CHUNK, DIMP), jnp.float32),
            pltpu.SemaphoreType.DMA((NBUF,)),
            pltpu.SemaphoreType.DMA((NBUF,)),
        ],
        compiler_params=pltpu.CompilerParams(use_tc_tiling_on_sc=False),
    )
    out128 = k(tpad, idx2d)
    return out128[:, :DIM]


# speedup vs baseline: 1.9318x; 1.3787x over previous
import jax
import jax.numpy as jnp
from jax import lax
from jax.experimental import pallas as pl
from jax.experimental.pallas import tpu as pltpu
from jax.experimental.pallas import tpu_sc as plsc

NUM_ROWS = 1_000_000
DIM = 64
BATCH = 425_984

NC = 2
NS = 16
NW = NC * NS
CHUNK = 128
DIMP = 128
ROWS_PER_W = BATCH // NW
CHUNKS_PER_W = ROWS_PER_W // CHUNK


NBUF = 4
NGROUPS = CHUNKS_PER_W // NBUF


def _gather_body(table_hbm, idx_hbm, out_hbm, idx_v, rows_v, gsem, wsem):
    wid = lax.axis_index("s") * NC + lax.axis_index("c")
    pltpu.sync_copy(idx_hbm.at[pl.ds(wid * CHUNKS_PER_W, CHUNKS_PER_W)], idx_v)
    base = wid * ROWS_PER_W

    for b in range(NBUF):
        pltpu.async_copy(table_hbm.at[idx_v.at[b]], rows_v.at[b], gsem.at[b])

    @pl.loop(0, NGROUPS - 1)
    def _group(g):
        for b in range(NBUF):
            j = g * NBUF + b
            pltpu.make_async_copy(
                table_hbm.at[idx_v.at[j]], rows_v.at[b], gsem.at[b]
            ).wait()
            pltpu.async_copy(
                rows_v.at[b],
                out_hbm.at[pl.ds(base + j * CHUNK, CHUNK)],
                wsem.at[b],
            ).wait()
            pltpu.async_copy(
                table_hbm.at[idx_v.at[j + NBUF]], rows_v.at[b], gsem.at[b]
            )

    for b in range(NBUF):
        j = (NGROUPS - 1) * NBUF + b
        pltpu.make_async_copy(
            table_hbm.at[idx_v.at[j]], rows_v.at[b], gsem.at[b]
        ).wait()
        pltpu.sync_copy(rows_v.at[b], out_hbm.at[pl.ds(base + j * CHUNK, CHUNK)])


TR_BLOCK = 8192
TR_GRID = (NUM_ROWS + TR_BLOCK - 1) // TR_BLOCK


def _transpose_body(in_ref, out_ref):
    out_ref[:, :DIM] = in_ref[...].T


def _transpose_pad(tT):
    return pl.pallas_call(
        _transpose_body,
        grid=(TR_GRID,),
        in_specs=[pl.BlockSpec((DIM, TR_BLOCK), lambda i: (0, i))],
        out_specs=pl.BlockSpec((TR_BLOCK, DIMP), lambda i: (i, 0)),
        out_shape=jax.ShapeDtypeStruct((NUM_ROWS, DIMP), jnp.float32),
    )(tT)


def kernel(tensor, index):
    tpad = _transpose_pad(jnp.swapaxes(tensor, 0, 1))
    idx2d = index.astype(jnp.int32).reshape(BATCH // CHUNK, CHUNK)
    mesh = plsc.VectorSubcoreMesh(core_axis_name="c", subcore_axis_name="s")
    k = pl.kernel(
        _gather_body,
        out_type=jax.ShapeDtypeStruct((BATCH, DIMP), jnp.float32),
        mesh=mesh,
        scratch_types=[
            pltpu.VMEM((CHUNKS_PER_W, CHUNK), jnp.int32),
            pltpu.VMEM((NBUF, CHUNK, DIMP), jnp.float32),
            pltpu.SemaphoreType.DMA((NBUF,)),
            pltpu.SemaphoreType.DMA((NBUF,)),
        ],
        compiler_params=pltpu.CompilerParams(use_tc_tiling_on_sc=False),
    )
    out128 = k(tpad, idx2d)
    return out128[:, :DIM]


# speedup vs baseline: 2.0102x; 1.0406x over previous
import jax
import jax.numpy as jnp
from jax import lax
from jax.experimental import pallas as pl
from jax.experimental.pallas import tpu as pltpu
from jax.experimental.pallas import tpu_sc as plsc

NUM_ROWS = 1_000_000
DIM = 64
BATCH = 425_984

NC = 2
NS = 16
NW = NC * NS
CHUNK = 128
DIMP = 128
ROWS_PER_W = BATCH // NW
CHUNKS_PER_W = ROWS_PER_W // CHUNK


NBUF = 4
NGROUPS = CHUNKS_PER_W // NBUF


def _gather_body(table_hbm, idx_hbm, out_hbm, idx_v, rows_v, gsem, wsem):
    wid = lax.axis_index("s") * NC + lax.axis_index("c")
    pltpu.sync_copy(idx_hbm.at[pl.ds(wid * CHUNKS_PER_W, CHUNKS_PER_W)], idx_v)
    base = wid * ROWS_PER_W

    for b in range(NBUF):
        pltpu.async_copy(table_hbm.at[idx_v.at[b]], rows_v.at[b], gsem.at[b])

    @pl.loop(0, NGROUPS - 1)
    def _group(g):
        for b in range(NBUF):
            j = g * NBUF + b
            pltpu.make_async_copy(
                table_hbm.at[idx_v.at[j]], rows_v.at[b], gsem.at[b]
            ).wait()
            pltpu.async_copy(
                rows_v.at[b],
                out_hbm.at[pl.ds(base + j * CHUNK, CHUNK)],
                wsem.at[b],
            ).wait()
            pltpu.async_copy(
                table_hbm.at[idx_v.at[j + NBUF]], rows_v.at[b], gsem.at[b]
            )

    for b in range(NBUF):
        j = (NGROUPS - 1) * NBUF + b
        pltpu.make_async_copy(
            table_hbm.at[idx_v.at[j]], rows_v.at[b], gsem.at[b]
        ).wait()
        pltpu.sync_copy(rows_v.at[b], out_hbm.at[pl.ds(base + j * CHUNK, CHUNK)])


TR_BLOCK = 16384
TR_GRID = (NUM_ROWS + TR_BLOCK - 1) // TR_BLOCK


def _transpose_body(in_ref, out_ref):
    out_ref[:, :DIM] = in_ref[...].T


def _transpose_pad(tT):
    return pl.pallas_call(
        _transpose_body,
        grid=(TR_GRID,),
        in_specs=[pl.BlockSpec((DIM, TR_BLOCK), lambda i: (0, i))],
        out_specs=pl.BlockSpec((TR_BLOCK, DIMP), lambda i: (i, 0)),
        out_shape=jax.ShapeDtypeStruct((NUM_ROWS, DIMP), jnp.float32),
    )(tT)


def kernel(tensor, index):
    tpad = _transpose_pad(jnp.swapaxes(tensor, 0, 1))
    idx2d = index.astype(jnp.int32).reshape(BATCH // CHUNK, CHUNK)
    mesh = plsc.VectorSubcoreMesh(core_axis_name="c", subcore_axis_name="s")
    k = pl.kernel(
        _gather_body,
        out_type=jax.ShapeDtypeStruct((BATCH, DIMP), jnp.float32),
        mesh=mesh,
        scratch_types=[
            pltpu.VMEM((CHUNKS_PER_W, CHUNK), jnp.int32),
            pltpu.VMEM((NBUF, CHUNK, DIMP), jnp.float32),
            pltpu.SemaphoreType.DMA((NBUF,)),
            pltpu.SemaphoreType.DMA((NBUF,)),
        ],
        compiler_params=pltpu.CompilerParams(use_tc_tiling_on_sc=False),
    )
    out128 = k(tpad, idx2d)
    return out128[:, :DIM]


# speedup vs baseline: 2.1990x; 1.0939x over previous
import jax
import jax.numpy as jnp
from jax import lax
from jax.experimental import pallas as pl
from jax.experimental.pallas import tpu as pltpu
from jax.experimental.pallas import tpu_sc as plsc

NUM_ROWS = 1_000_000
DIM = 64
BATCH = 425_984

NC = 2
NS = 16
NW = NC * NS
CHUNK = 128
DIMP = 128
ROWS_PER_W = BATCH // NW
CHUNKS_PER_W = ROWS_PER_W // CHUNK


NBUF = 4
NGROUPS = CHUNKS_PER_W // NBUF


def _gather_body(table_hbm, idx_hbm, out_hbm, idx_v, rows_v, gsem, wsem):
    wid = lax.axis_index("s") * NC + lax.axis_index("c")
    pltpu.sync_copy(idx_hbm.at[pl.ds(wid * CHUNKS_PER_W, CHUNKS_PER_W)], idx_v)
    base = wid * ROWS_PER_W

    for b in range(NBUF):
        pltpu.async_copy(table_hbm.at[idx_v.at[b]], rows_v.at[b], gsem.at[b])

    @pl.loop(0, NGROUPS - 1)
    def _group(g):
        for b in range(NBUF):
            j = g * NBUF + b
            pltpu.make_async_copy(
                table_hbm.at[idx_v.at[j]], rows_v.at[b], gsem.at[b]
            ).wait()
            pltpu.async_copy(
                rows_v.at[b].at[:, pl.ds(0, DIM)],
                out_hbm.at[pl.ds(base + j * CHUNK, CHUNK), pl.ds(0, DIM)],
                wsem.at[b],
            ).wait()
            pltpu.async_copy(
                table_hbm.at[idx_v.at[j + NBUF]], rows_v.at[b], gsem.at[b]
            )

    for b in range(NBUF):
        j = (NGROUPS - 1) * NBUF + b
        pltpu.make_async_copy(
            table_hbm.at[idx_v.at[j]], rows_v.at[b], gsem.at[b]
        ).wait()
        pltpu.sync_copy(
            rows_v.at[b].at[:, pl.ds(0, DIM)],
            out_hbm.at[pl.ds(base + j * CHUNK, CHUNK), pl.ds(0, DIM)],
        )


TR_BLOCK = 32768
TR_GRID = (NUM_ROWS + TR_BLOCK - 1) // TR_BLOCK


def _transpose_body(in_ref, out_ref):
    out_ref[:, :DIM] = in_ref[...].T


def _transpose_pad(tT):
    return pl.pallas_call(
        _transpose_body,
        grid=(TR_GRID,),
        in_specs=[pl.BlockSpec((DIM, TR_BLOCK), lambda i: (0, i))],
        out_specs=pl.BlockSpec((TR_BLOCK, DIMP), lambda i: (i, 0)),
        out_shape=jax.ShapeDtypeStruct((NUM_ROWS, DIMP), jnp.float32),
    )(tT)


def kernel(tensor, index):
    tpad = _transpose_pad(jnp.swapaxes(tensor, 0, 1))
    idx2d = index.astype(jnp.int32).reshape(BATCH // CHUNK, CHUNK)
    mesh = plsc.VectorSubcoreMesh(core_axis_name="c", subcore_axis_name="s")
    k = pl.kernel(
        _gather_body,
        out_type=jax.ShapeDtypeStruct((BATCH, DIMP), jnp.float32),
        mesh=mesh,
        scratch_types=[
            pltpu.VMEM((CHUNKS_PER_W, CHUNK), jnp.int32),
            pltpu.VMEM((NBUF, CHUNK, DIMP), jnp.float32),
            pltpu.SemaphoreType.DMA((NBUF,)),
            pltpu.SemaphoreType.DMA((NBUF,)),
        ],
        compiler_params=pltpu.CompilerParams(use_tc_tiling_on_sc=False),
    )
    out128 = k(tpad, idx2d)
    return out128[:, :DIM]


# speedup vs baseline: 2.2868x; 1.0399x over previous
import jax
import jax.numpy as jnp
from jax import lax
from jax.experimental import pallas as pl
from jax.experimental.pallas import tpu as pltpu
from jax.experimental.pallas import tpu_sc as plsc

NUM_ROWS = 1_000_000
DIM = 64
BATCH = 425_984

NC = 2
NS = 16
NW = NC * NS
CHUNK = 128
DIMP = 128
ROWS_PER_W = BATCH // NW
CHUNKS_PER_W = ROWS_PER_W // CHUNK


NBUF = 8
NGROUPS = CHUNKS_PER_W // NBUF


def _gather_body(table_hbm, idx_hbm, out_hbm, idx_v, rows_v, gsem, wsem):
    wid = lax.axis_index("s") * NC + lax.axis_index("c")
    pltpu.sync_copy(idx_hbm.at[pl.ds(wid * CHUNKS_PER_W, CHUNKS_PER_W)], idx_v)
    base = wid * ROWS_PER_W

    for b in range(NBUF):
        pltpu.async_copy(table_hbm.at[idx_v.at[b]], rows_v.at[b], gsem.at[b])

    @pl.loop(0, NGROUPS - 1)
    def _group(g):
        for b in range(NBUF):
            j = g * NBUF + b
            pltpu.make_async_copy(
                table_hbm.at[idx_v.at[j]], rows_v.at[b], gsem.at[b]
            ).wait()
            pltpu.async_copy(
                rows_v.at[b],
                out_hbm.at[pl.ds(base + j * CHUNK, CHUNK), pl.ds(0, DIM)],
                wsem.at[b],
            ).wait()
            pltpu.async_copy(
                table_hbm.at[idx_v.at[j + NBUF]], rows_v.at[b], gsem.at[b]
            )

    for b in range(NBUF):
        j = (NGROUPS - 1) * NBUF + b
        pltpu.make_async_copy(
            table_hbm.at[idx_v.at[j]], rows_v.at[b], gsem.at[b]
        ).wait()
        pltpu.sync_copy(
            rows_v.at[b],
            out_hbm.at[pl.ds(base + j * CHUNK, CHUNK), pl.ds(0, DIM)],
        )


KPACK = 512_000
TR_BLOCK = 4096
TR_GRID = KPACK // TR_BLOCK
TR_OFF = TR_GRID
TR_LAST = (NUM_ROWS - 1) // TR_BLOCK


def _transpose_body(top_ref, bot_ref, out_ref):
    out_ref[:, :DIM] = top_ref[...].T
    out_ref[:, DIM:] = bot_ref[...].T


def _transpose_pack(tT):
    return pl.pallas_call(
        _transpose_body,
        grid=(TR_GRID,),
        in_specs=[
            pl.BlockSpec((DIM, TR_BLOCK), lambda i: (0, i)),
            pl.BlockSpec(
                (DIM, TR_BLOCK), lambda i: (0, jnp.minimum(i + TR_OFF, TR_LAST))
            ),
        ],
        out_specs=pl.BlockSpec((TR_BLOCK, DIMP), lambda i: (i, 0)),
        out_shape=jax.ShapeDtypeStruct((KPACK, DIMP), jnp.float32),
    )(tT, tT)


def kernel(tensor, index):
    tpack = _transpose_pack(jnp.swapaxes(tensor, 0, 1))
    tbl = tpack.reshape(2 * KPACK, DIM)
    idx = index.astype(jnp.int32)
    lidx = jnp.where(idx < KPACK, 2 * idx, 2 * (idx - KPACK) + 1)
    idx2d = lidx.reshape(BATCH // CHUNK, CHUNK)
    mesh = plsc.VectorSubcoreMesh(core_axis_name="c", subcore_axis_name="s")
    k = pl.kernel(
        _gather_body,
        out_type=jax.ShapeDtypeStruct((BATCH, DIMP), jnp.float32),
        mesh=mesh,
        scratch_types=[
            pltpu.VMEM((CHUNKS_PER_W, CHUNK), jnp.int32),
            pltpu.VMEM((NBUF, CHUNK, DIM), jnp.float32),
            pltpu.SemaphoreType.DMA((NBUF,)),
            pltpu.SemaphoreType.DMA((NBUF,)),
        ],
        compiler_params=pltpu.CompilerParams(use_tc_tiling_on_sc=False),
    )
    out128 = k(tbl, idx2d)
    return out128[:, :DIM]


# speedup vs baseline: 2.5615x; 1.1201x over previous
import jax
import jax.numpy as jnp
from jax import lax
from jax.experimental import pallas as pl
from jax.experimental.pallas import tpu as pltpu
from jax.experimental.pallas import tpu_sc as plsc

NUM_ROWS = 1_000_000
DIM = 64
BATCH = 425_984

NC = 2
NS = 16
NW = NC * NS
CHUNK = 128
DIMP = 128
ROWS_PER_W = BATCH // NW
CHUNKS_PER_W = ROWS_PER_W // CHUNK


NBUF = 8
NGROUPS = CHUNKS_PER_W // NBUF


def _gather_body(table_hbm, idx_hbm, out_hbm, idx_v, rows_v, gsem, wsem):
    wid = lax.axis_index("s") * NC + lax.axis_index("c")
    pltpu.sync_copy(idx_hbm.at[pl.ds(wid * CHUNKS_PER_W, CHUNKS_PER_W)], idx_v)
    base = wid * ROWS_PER_W

    for b in range(NBUF):
        pltpu.async_copy(table_hbm.at[idx_v.at[b]], rows_v.at[b], gsem.at[b])

    @pl.loop(0, NGROUPS - 1)
    def _group(g):
        for b in range(NBUF):
            j = g * NBUF + b
            pltpu.make_async_copy(
                table_hbm.at[idx_v.at[j]], rows_v.at[b], gsem.at[b]
            ).wait()
            pltpu.async_copy(
                rows_v.at[b],
                out_hbm.at[pl.ds(base + j * CHUNK, CHUNK), pl.ds(0, DIM)],
                wsem.at[b],
            ).wait()
            pltpu.async_copy(
                table_hbm.at[idx_v.at[j + NBUF]], rows_v.at[b], gsem.at[b]
            )

    for b in range(NBUF):
        j = (NGROUPS - 1) * NBUF + b
        pltpu.make_async_copy(
            table_hbm.at[idx_v.at[j]], rows_v.at[b], gsem.at[b]
        ).wait()
        pltpu.sync_copy(
            rows_v.at[b],
            out_hbm.at[pl.ds(base + j * CHUNK, CHUNK), pl.ds(0, DIM)],
        )


KPACK = 512_000
TR_BLOCK = 16000
TR_GRID = KPACK // TR_BLOCK
TR_OFF = TR_GRID
TR_LAST = (NUM_ROWS - 1) // TR_BLOCK


def _transpose_body(top_ref, bot_ref, out_ref):
    out_ref[:, :DIM] = top_ref[...].T
    out_ref[:, DIM:] = bot_ref[...].T


def _transpose_pack(tT):
    return pl.pallas_call(
        _transpose_body,
        grid=(TR_GRID,),
        in_specs=[
            pl.BlockSpec((DIM, TR_BLOCK), lambda i: (0, i)),
            pl.BlockSpec(
                (DIM, TR_BLOCK), lambda i: (0, jnp.minimum(i + TR_OFF, TR_LAST))
            ),
        ],
        out_specs=pl.BlockSpec((TR_BLOCK, DIMP), lambda i: (i, 0)),
        out_shape=jax.ShapeDtypeStruct((KPACK, DIMP), jnp.float32),
    )(tT, tT)


def kernel(tensor, index):
    tpack = _transpose_pack(jnp.swapaxes(tensor, 0, 1))
    tbl = tpack.reshape(2 * KPACK, DIM)
    idx = index.astype(jnp.int32)
    lidx = jnp.where(idx < KPACK, 2 * idx, 2 * (idx - KPACK) + 1)
    idx2d = lidx.reshape(BATCH // CHUNK, CHUNK)
    mesh = plsc.VectorSubcoreMesh(core_axis_name="c", subcore_axis_name="s")
    k = pl.kernel(
        _gather_body,
        out_type=jax.ShapeDtypeStruct((BATCH, DIMP), jnp.float32),
        mesh=mesh,
        scratch_types=[
            pltpu.VMEM((CHUNKS_PER_W, CHUNK), jnp.int32),
            pltpu.VMEM((NBUF, CHUNK, DIM), jnp.float32),
            pltpu.SemaphoreType.DMA((NBUF,)),
            pltpu.SemaphoreType.DMA((NBUF,)),
        ],
        compiler_params=pltpu.CompilerParams(use_tc_tiling_on_sc=False),
    )
    out128 = k(tbl, idx2d)
    return out128[:, :DIM]
